# baseline (speedup 1.0000x reference)
import functools

import jax
import jax.numpy as jnp
from jax.experimental import pallas as pl
from jax.experimental.pallas import tpu as pltpu

_A = 128
_S = 4
_SB = 2048


def _mm_body(mt_ref, bs_ref, xs_ref, mk_ref, *rest):
    q_refs = rest[:_S]
    p_ref = rest[_S]
    A, C = xs_ref.shape
    x = xs_ref[...]
    accs = []
    for s in range(_S):
        xm = x * mk_ref[0, :, s:s + 1]
        accs.append(jax.lax.dot_general(
            xm, q_refs[s][0], (((1,), (0,)), ((), ())),
            precision=jax.lax.Precision.HIGHEST,
            preferred_element_type=jnp.float32))
    while len(accs) > 1:
        accs = [a + b for a, b in zip(accs[::2], accs[1::2])]
    union = mk_ref[0, :, _S:_S + 1] > 0.0
    p_ref[...] = jnp.where(union, accs[0], p_ref[...])


def _sample_body(p_ref, g_ref, o_ref):
    B, C = p_ref.shape
    p = p_ref[...]
    ssum = jnp.sum(p, axis=-1, keepdims=True)
    logit = jnp.log(jnp.maximum(p / ssum, 1e-30)) + g_ref[...]
    mx = jnp.max(logit, axis=-1, keepdims=True)
    iot = jax.lax.broadcasted_iota(jnp.int32, (B, C), 1)
    idx = jnp.min(jnp.where(logit == mx, iot, C), axis=-1, keepdims=True)
    o_ref[...] = (iot == idx).astype(jnp.float32)


def kernel(x0_batch, time_batch, accumulated_q_matrices):
    N, C = x0_batch.shape
    TQ = accumulated_q_matrices.shape[0]
    A, S = _A, _S
    NB = N // A
    MCAP = TQ + NB
    GCAP = NB + (MCAP + S - 1) // S

    t = time_batch.astype(jnp.int32)
    order = jnp.argsort(t)
    ts = t[order]
    xs = x0_batch[order]

    iota = jnp.arange(N, dtype=jnp.int32)
    new_t = jnp.concatenate(
        [jnp.ones((1,), jnp.bool_), ts[1:] != ts[:-1]])
    seg_begin = new_t | ((iota % A) == 0)
    start_rows = jnp.nonzero(seg_begin, size=MCAP, fill_value=N)[0].astype(jnp.int32)
    valid = start_rows < N
    nxt = jnp.concatenate([start_rows[1:], jnp.full((1,), N, jnp.int32)])
    end_rows = jnp.where(valid, nxt, N)
    seg_t = ts[jnp.clip(start_rows, 0, N - 1)]
    blk = jnp.clip(start_rows // A, 0, NB - 1)
    first_m = jnp.searchsorted(
        start_rows, jnp.arange(NB, dtype=jnp.int32) * A).astype(jnp.int32)
    mtot = jnp.sum(seg_begin.astype(jnp.int32))
    first_ext = jnp.concatenate([first_m, mtot[None]])
    count = first_ext[1:] - first_ext[:-1]
    steps_b = (count + S - 1) // S
    step_off = jnp.concatenate(
        [jnp.zeros((1,), jnp.int32), jnp.cumsum(steps_b, dtype=jnp.int32)])
    m_idx = jnp.arange(MCAP, dtype=jnp.int32)
    pos = m_idx - first_m[blk]
    gstep = step_off[blk] + pos // S
    slot = pos % S
    flat = jnp.where(valid, gstep * S + slot, GCAP * S)
    meta_start = jnp.full((GCAP * S,), A, jnp.int32).at[flat].set(
        start_rows - blk * A, mode='drop')
    meta_end = jnp.zeros((GCAP * S,), jnp.int32).at[flat].set(
        end_rows - blk * A, mode='drop')
    mt = jnp.full((GCAP * S,), -1, jnp.int32).at[flat].set(
        seg_t, mode='drop').reshape(GCAP, S)
    gi = jnp.arange(GCAP, dtype=jnp.int32)[:, None]
    last = jax.lax.cummax(jnp.where(mt >= 0, gi, -1), axis=0)
    mt_ff = jnp.take_along_axis(mt, jnp.clip(last, 0, None), axis=0)
    meta_time = jnp.clip(mt_ff, 0, TQ - 1).reshape(-1)
    blk_step = jnp.full((GCAP,), NB - 1, jnp.int32).at[
        jnp.where(valid, gstep, GCAP)].set(blk, mode='drop')

    rr = jnp.arange(A, dtype=jnp.int32).reshape(1, A, 1)
    ms3 = meta_start.reshape(GCAP, 1, S)
    me3 = meta_end.reshape(GCAP, 1, S)
    m3 = ((rr >= ms3) & (rr < me3)).astype(jnp.float32)
    union3 = jnp.sum(m3, axis=-1, keepdims=True)
    masks = jnp.concatenate(
        [m3, union3, jnp.zeros((GCAP, A, 16 - S - 1), jnp.float32)], axis=-1)

    q_specs = [
        pl.BlockSpec((1, C, C), functools.partial(
            lambda i, mt_, bs, s: (mt_[i * S + s], 0, 0), s=s))
        for s in range(S)
    ]
    grid_spec = pltpu.PrefetchScalarGridSpec(
        num_scalar_prefetch=2,
        grid=(GCAP,),
        in_specs=[
            pl.BlockSpec((A, C), lambda i, mt_, bs: (bs[i], 0)),
            pl.BlockSpec((1, A, 16), lambda i, mt_, bs: (i, 0, 0)),
            *q_specs,
        ],
        out_specs=[
            pl.BlockSpec((A, C), lambda i, mt_, bs: (bs[i], 0)),
        ],
    )
    ps, = pl.pallas_call(
        _mm_body,
        grid_spec=grid_spec,
        out_shape=[jax.ShapeDtypeStruct((N, C), jnp.float32)],
    )(meta_time, blk_step, xs, masks, *([accumulated_q_matrices] * S))

    inv = jnp.zeros((N,), jnp.int32).at[order].set(iota)
    probs = ps[inv]

    gum = jax.random.gumbel(jax.random.key(1), (N, C), jnp.float32)
    noised = pl.pallas_call(
        _sample_body,
        grid=(N // _SB,),
        in_specs=[
            pl.BlockSpec((_SB, C), lambda i: (i, 0)),
            pl.BlockSpec((_SB, C), lambda i: (i, 0)),
        ],
        out_specs=pl.BlockSpec((_SB, C), lambda i: (i, 0)),
        out_shape=jax.ShapeDtypeStruct((N, C), jnp.float32),
    )(probs, gum)
    return probs, noised


# speedup vs baseline: 1.0606x; 1.0606x over previous
import functools

import jax
import jax.numpy as jnp
from jax.experimental import pallas as pl
from jax.experimental.pallas import tpu as pltpu

_A = 128
_S = 16
_SB = 2048


def _mm_body(mt_ref, bs_ref, xs_ref, mk_ref, *rest):
    q_refs = rest[:_S]
    p_ref = rest[_S]
    A, C = xs_ref.shape
    x = xs_ref[...]
    accs = []
    for s in range(_S):
        xm = x * mk_ref[0, :, s:s + 1]
        accs.append(jax.lax.dot_general(
            xm, q_refs[s][0], (((1,), (0,)), ((), ())),
            precision=jax.lax.Precision.HIGHEST,
            preferred_element_type=jnp.float32))
    while len(accs) > 1:
        accs = [a + b for a, b in zip(accs[::2], accs[1::2])]
    union = mk_ref[0, :, _S:_S + 1] > 0.0
    p_ref[...] = jnp.where(union, accs[0], p_ref[...])


def _sample_body(p_ref, g_ref, o_ref):
    B, C = p_ref.shape
    p = p_ref[...]
    ssum = jnp.sum(p, axis=-1, keepdims=True)
    logit = jnp.log(jnp.maximum(p / ssum, 1e-30)) + g_ref[...]
    mx = jnp.max(logit, axis=-1, keepdims=True)
    iot = jax.lax.broadcasted_iota(jnp.int32, (B, C), 1)
    idx = jnp.min(jnp.where(logit == mx, iot, C), axis=-1, keepdims=True)
    o_ref[...] = (iot == idx).astype(jnp.float32)


def kernel(x0_batch, time_batch, accumulated_q_matrices):
    N, C = x0_batch.shape
    TQ = accumulated_q_matrices.shape[0]
    A, S = _A, _S
    NB = N // A
    MCAP = TQ + NB
    GCAP = NB + (MCAP + S - 1) // S

    t = time_batch.astype(jnp.int32)
    order = jnp.argsort(t)
    ts = t[order]
    xs = x0_batch[order]

    iota = jnp.arange(N, dtype=jnp.int32)
    new_t = jnp.concatenate(
        [jnp.ones((1,), jnp.bool_), ts[1:] != ts[:-1]])
    seg_begin = new_t | ((iota % A) == 0)
    start_rows = jnp.nonzero(seg_begin, size=MCAP, fill_value=N)[0].astype(jnp.int32)
    valid = start_rows < N
    nxt = jnp.concatenate([start_rows[1:], jnp.full((1,), N, jnp.int32)])
    end_rows = jnp.where(valid, nxt, N)
    seg_t = ts[jnp.clip(start_rows, 0, N - 1)]
    blk = jnp.clip(start_rows // A, 0, NB - 1)
    first_m = jnp.searchsorted(
        start_rows, jnp.arange(NB, dtype=jnp.int32) * A).astype(jnp.int32)
    mtot = jnp.sum(seg_begin.astype(jnp.int32))
    first_ext = jnp.concatenate([first_m, mtot[None]])
    count = first_ext[1:] - first_ext[:-1]
    steps_b = (count + S - 1) // S
    step_off = jnp.concatenate(
        [jnp.zeros((1,), jnp.int32), jnp.cumsum(steps_b, dtype=jnp.int32)])
    m_idx = jnp.arange(MCAP, dtype=jnp.int32)
    pos = m_idx - first_m[blk]
    gstep = step_off[blk] + pos // S
    slot = pos % S
    flat = jnp.where(valid, gstep * S + slot, GCAP * S)
    meta_start = jnp.full((GCAP * S,), A, jnp.int32).at[flat].set(
        start_rows - blk * A, mode='drop')
    meta_end = jnp.zeros((GCAP * S,), jnp.int32).at[flat].set(
        end_rows - blk * A, mode='drop')
    mt = jnp.full((GCAP * S,), -1, jnp.int32).at[flat].set(
        seg_t, mode='drop').reshape(GCAP, S)
    gi = jnp.arange(GCAP, dtype=jnp.int32)[:, None]
    last = jax.lax.cummax(jnp.where(mt >= 0, gi, -1), axis=0)
    mt_ff = jnp.take_along_axis(mt, jnp.clip(last, 0, None), axis=0)
    meta_time = jnp.clip(mt_ff, 0, TQ - 1).reshape(-1)
    blk_step = jnp.full((GCAP,), NB - 1, jnp.int32).at[
        jnp.where(valid, gstep, GCAP)].set(blk, mode='drop')

    rr = jnp.arange(A, dtype=jnp.int32).reshape(1, A, 1)
    ms3 = meta_start.reshape(GCAP, 1, S)
    me3 = meta_end.reshape(GCAP, 1, S)
    m3 = ((rr >= ms3) & (rr < me3)).astype(jnp.float32)
    union3 = jnp.sum(m3, axis=-1, keepdims=True)
    masks = jnp.concatenate(
        [m3, union3, jnp.zeros((GCAP, A, 32 - S - 1), jnp.float32)], axis=-1)

    q_specs = [
        pl.BlockSpec((1, C, C), functools.partial(
            lambda i, mt_, bs, s: (mt_[i * S + s], 0, 0), s=s))
        for s in range(S)
    ]
    grid_spec = pltpu.PrefetchScalarGridSpec(
        num_scalar_prefetch=2,
        grid=(GCAP,),
        in_specs=[
            pl.BlockSpec((A, C), lambda i, mt_, bs: (bs[i], 0)),
            pl.BlockSpec((1, A, 32), lambda i, mt_, bs: (i, 0, 0)),
            *q_specs,
        ],
        out_specs=[
            pl.BlockSpec((A, C), lambda i, mt_, bs: (bs[i], 0)),
        ],
    )
    ps, = pl.pallas_call(
        _mm_body,
        grid_spec=grid_spec,
        out_shape=[jax.ShapeDtypeStruct((N, C), jnp.float32)],
    )(meta_time, blk_step, xs, masks, *([accumulated_q_matrices] * S))

    inv = jnp.zeros((N,), jnp.int32).at[order].set(iota)
    probs = ps[inv]

    gum = jax.random.gumbel(jax.random.key(1), (N, C), jnp.float32)
    noised = pl.pallas_call(
        _sample_body,
        grid=(N // _SB,),
        in_specs=[
            pl.BlockSpec((_SB, C), lambda i: (i, 0)),
            pl.BlockSpec((_SB, C), lambda i: (i, 0)),
        ],
        out_specs=pl.BlockSpec((_SB, C), lambda i: (i, 0)),
        out_shape=jax.ShapeDtypeStruct((N, C), jnp.float32),
    )(probs, gum)
    return probs, noised


# speedup vs baseline: 1.1672x; 1.1004x over previous
import functools

import jax
import jax.numpy as jnp
from jax.experimental import pallas as pl
from jax.experimental.pallas import tpu as pltpu

_A = 128
_S = 8
_SB = 2048


def _mm_body(mt_ref, bs_ref, xs_ref, mk_ref, *rest):
    q_refs = rest[:_S]
    p_ref = rest[_S]
    A, C = xs_ref.shape
    x = xs_ref[...]
    accs = []
    for s in range(_S):
        xm = x * mk_ref[0, :, s:s + 1]
        accs.append(jax.lax.dot_general(
            xm, q_refs[s][0], (((1,), (0,)), ((), ())),
            precision=jax.lax.Precision.HIGHEST,
            preferred_element_type=jnp.float32))
    while len(accs) > 1:
        accs = [a + b for a, b in zip(accs[::2], accs[1::2])]
    union = mk_ref[0, :, _S:_S + 1] > 0.0
    p_ref[...] = jnp.where(union, accs[0], p_ref[...])


def _sample_body(p_ref, g_ref, o_ref):
    B, C = p_ref.shape
    p = p_ref[...]
    ssum = jnp.sum(p, axis=-1, keepdims=True)
    logit = jnp.log(jnp.maximum(p / ssum, 1e-30)) + g_ref[...]
    mx = jnp.max(logit, axis=-1, keepdims=True)
    iot = jax.lax.broadcasted_iota(jnp.int32, (B, C), 1)
    idx = jnp.min(jnp.where(logit == mx, iot, C), axis=-1, keepdims=True)
    o_ref[...] = (iot == idx).astype(jnp.float32)


def kernel(x0_batch, time_batch, accumulated_q_matrices):
    N, C = x0_batch.shape
    TQ = accumulated_q_matrices.shape[0]
    A, S = _A, _S
    NB = N // A
    MCAP = TQ + NB
    GCAP = NB + (MCAP + S - 1) // S

    t = time_batch.astype(jnp.int32)
    iota0 = jnp.arange(N, dtype=jnp.int32)
    packed = jnp.sort(t * N + iota0)
    order = packed % N
    ts = packed // N
    xs = x0_batch[order]

    iota = jnp.arange(N, dtype=jnp.int32)
    new_t = jnp.concatenate(
        [jnp.ones((1,), jnp.bool_), ts[1:] != ts[:-1]])
    seg_begin = new_t | ((iota % A) == 0)
    start_rows = jnp.nonzero(seg_begin, size=MCAP, fill_value=N)[0].astype(jnp.int32)
    valid = start_rows < N
    nxt = jnp.concatenate([start_rows[1:], jnp.full((1,), N, jnp.int32)])
    end_rows = jnp.where(valid, nxt, N)
    seg_t = ts[jnp.clip(start_rows, 0, N - 1)]
    blk = jnp.clip(start_rows // A, 0, NB - 1)
    first_m = jnp.searchsorted(
        start_rows, jnp.arange(NB, dtype=jnp.int32) * A).astype(jnp.int32)
    mtot = jnp.sum(seg_begin.astype(jnp.int32))
    first_ext = jnp.concatenate([first_m, mtot[None]])
    count = first_ext[1:] - first_ext[:-1]
    steps_b = (count + S - 1) // S
    step_off = jnp.concatenate(
        [jnp.zeros((1,), jnp.int32), jnp.cumsum(steps_b, dtype=jnp.int32)])
    m_idx = jnp.arange(MCAP, dtype=jnp.int32)
    pos = m_idx - first_m[blk]
    gstep = step_off[blk] + pos // S
    slot = pos % S
    flat = jnp.where(valid, gstep * S + slot, GCAP * S)
    meta_start = jnp.full((GCAP * S,), A, jnp.int32).at[flat].set(
        start_rows - blk * A, mode='drop')
    meta_end = jnp.zeros((GCAP * S,), jnp.int32).at[flat].set(
        end_rows - blk * A, mode='drop')
    mt = jnp.full((GCAP * S,), -1, jnp.int32).at[flat].set(
        seg_t, mode='drop').reshape(GCAP, S)
    gi = jnp.arange(GCAP, dtype=jnp.int32)[:, None]
    last = jax.lax.cummax(jnp.where(mt >= 0, gi, -1), axis=0)
    mt_ff = jnp.take_along_axis(mt, jnp.clip(last, 0, None), axis=0)
    meta_time = jnp.clip(mt_ff, 0, TQ - 1).reshape(-1)
    blk_step = jnp.full((GCAP,), NB - 1, jnp.int32).at[
        jnp.where(valid, gstep, GCAP)].set(blk, mode='drop')

    rr = jnp.arange(A, dtype=jnp.int32).reshape(1, A, 1)
    ms3 = meta_start.reshape(GCAP, 1, S)
    me3 = meta_end.reshape(GCAP, 1, S)
    m3 = ((rr >= ms3) & (rr < me3)).astype(jnp.float32)
    union3 = jnp.sum(m3, axis=-1, keepdims=True)
    masks = jnp.concatenate(
        [m3, union3, jnp.zeros((GCAP, A, 16 - S - 1), jnp.float32)], axis=-1)

    q_specs = [
        pl.BlockSpec((1, C, C), functools.partial(
            lambda i, mt_, bs, s: (mt_[i * S + s], 0, 0), s=s))
        for s in range(S)
    ]
    grid_spec = pltpu.PrefetchScalarGridSpec(
        num_scalar_prefetch=2,
        grid=(GCAP,),
        in_specs=[
            pl.BlockSpec((A, C), lambda i, mt_, bs: (bs[i], 0)),
            pl.BlockSpec((1, A, 16), lambda i, mt_, bs: (i, 0, 0)),
            *q_specs,
        ],
        out_specs=[
            pl.BlockSpec((A, C), lambda i, mt_, bs: (bs[i], 0)),
        ],
    )
    ps, = pl.pallas_call(
        _mm_body,
        grid_spec=grid_spec,
        out_shape=[jax.ShapeDtypeStruct((N, C), jnp.float32)],
    )(meta_time, blk_step, xs, masks, *([accumulated_q_matrices] * S))

    inv = jnp.zeros((N,), jnp.int32).at[order].set(iota)
    probs = ps[inv]

    gum = jax.random.gumbel(jax.random.key(1), (N, C), jnp.float32)
    noised = pl.pallas_call(
        _sample_body,
        grid=(N // _SB,),
        in_specs=[
            pl.BlockSpec((_SB, C), lambda i: (i, 0)),
            pl.BlockSpec((_SB, C), lambda i: (i, 0)),
        ],
        out_specs=pl.BlockSpec((_SB, C), lambda i: (i, 0)),
        out_shape=jax.ShapeDtypeStruct((N, C), jnp.float32),
    )(probs, gum)
    return probs, noised
